# baseline (speedup 1.0000x reference)
import jax
import jax.numpy as jnp
from jax.experimental import pallas as pl
from jax.experimental.pallas import tpu as pltpu

_BR = 200
_NBUF = 4
_N = 10000
_NSTEP = _N // _BR


def _body(adj_hbm, xk_ref, w1a_ref, w1b_ref, b1_ref,
          w2_ref, b2_ref, pa_ref, w3_ref, b3_ref, out_ref,
          bufs, sems):
    i = pl.program_id(0)

    def copy(j, slot):
        return pltpu.make_async_copy(
            adj_hbm.at[pl.ds(j * _BR, _BR), :], bufs.at[slot], sems.at[slot])

    @pl.when(i == 0)
    def _():
        for k in range(_NBUF - 1):
            copy(k, k).start()

    slot = jax.lax.rem(i, _NBUF)
    copy(i, slot).wait()

    nxt = i + _NBUF - 1
    @pl.when(nxt < _NSTEP)
    def _():
        copy(nxt, jax.lax.rem(nxt, _NBUF)).start()

    agg = jnp.dot(bufs[slot], xk_ref[...], preferred_element_type=jnp.float32)
    xr = xk_ref[pl.ds(i * _BR, _BR), :]
    h = jnp.dot(xr, w1a_ref[...], preferred_element_type=jnp.float32)
    h += jnp.dot(agg, w1b_ref[...], preferred_element_type=jnp.float32)
    h = jnp.maximum(h + b1_ref[...], 0.0)
    z = jnp.dot(h, w2_ref[...], preferred_element_type=jnp.float32) + b2_ref[...]
    p = jnp.where(z >= 0, z, pa_ref[...] * z)
    out_ref[...] = jnp.dot(p, w3_ref[...],
                           preferred_element_type=jnp.float32) + b3_ref[...]


@jax.jit
def kernel(x, adj, W1, b1, W2, b2, prelu_a, W3, b3):
    n, d = x.shape
    nhid = W2.shape[0]
    nclass = W3.shape[1]

    w1a = W1[:d]
    w1b = W1[d:]
    b1r = b1.reshape(1, nhid)
    b2r = b2.reshape(1, nhid)
    par = prelu_a.reshape(1, nhid)
    b3r = b3.reshape(1, nclass)

    grid = (_NSTEP,)
    out = pl.pallas_call(
        _body,
        grid=grid,
        in_specs=[
            pl.BlockSpec(memory_space=pltpu.MemorySpace.HBM),
            pl.BlockSpec((n, d), lambda i: (0, 0)),
            pl.BlockSpec((d, nhid), lambda i: (0, 0)),
            pl.BlockSpec((d, nhid), lambda i: (0, 0)),
            pl.BlockSpec((1, nhid), lambda i: (0, 0)),
            pl.BlockSpec((nhid, nhid), lambda i: (0, 0)),
            pl.BlockSpec((1, nhid), lambda i: (0, 0)),
            pl.BlockSpec((1, nhid), lambda i: (0, 0)),
            pl.BlockSpec((nhid, nclass), lambda i: (0, 0)),
            pl.BlockSpec((1, nclass), lambda i: (0, 0)),
        ],
        out_specs=pl.BlockSpec((_BR, nclass), lambda i: (i, 0)),
        out_shape=jax.ShapeDtypeStruct((n, nclass), jnp.float32),
        scratch_shapes=[
            pltpu.VMEM((_NBUF, _BR, _N), jnp.float32),
            pltpu.SemaphoreType.DMA((_NBUF,)),
        ],
        compiler_params=pltpu.CompilerParams(
            dimension_semantics=("arbitrary",),
            vmem_limit_bytes=128 * 1024 * 1024),
    )(adj, x, w1a, w1b, b1r, W2, b2r, par, W3, b3r)
    return out


# speedup vs baseline: 1.1686x; 1.1686x over previous
import jax
import jax.numpy as jnp
from jax.experimental import pallas as pl
from jax.experimental.pallas import tpu as pltpu

_BR = 200
_BM = 2 * _BR
_NBUF = 3
_N = 10000
_NSTEP = _N // _BM


def _body(adj_a, adj_b, xk_ref, w1a_ref, w1b_ref, b1_ref,
          w2_ref, b2_ref, pa_ref, w3_ref, b3_ref, out_ref,
          bufs_a, bufs_b, sems_a, sems_b):
    i = pl.program_id(0)

    def copy_a(j, slot):
        return pltpu.make_async_copy(
            adj_a.at[pl.ds(j * _BM, _BR), :], bufs_a.at[slot], sems_a.at[slot])

    def copy_b(j, slot):
        return pltpu.make_async_copy(
            adj_b.at[pl.ds(j * _BM + _BR, _BR), :], bufs_b.at[slot],
            sems_b.at[slot])

    @pl.when(i == 0)
    def _():
        for k in range(_NBUF - 1):
            copy_a(k, k).start()
            copy_b(k, k).start()

    slot = jax.lax.rem(i, _NBUF)
    copy_a(i, slot).wait()
    copy_b(i, slot).wait()

    nxt = i + _NBUF - 1
    @pl.when(nxt < _NSTEP)
    def _():
        nslot = jax.lax.rem(nxt, _NBUF)
        copy_a(nxt, nslot).start()
        copy_b(nxt, nslot).start()

    agg_a = jnp.dot(bufs_a[slot], xk_ref[...],
                    preferred_element_type=jnp.float32)
    agg_b = jnp.dot(bufs_b[slot], xk_ref[...],
                    preferred_element_type=jnp.float32)
    agg = jnp.concatenate([agg_a, agg_b], axis=0)
    xr = xk_ref[pl.ds(i * _BM, _BM), :]
    h = jnp.dot(xr, w1a_ref[...], preferred_element_type=jnp.float32)
    h += jnp.dot(agg, w1b_ref[...], preferred_element_type=jnp.float32)
    h = jnp.maximum(h + b1_ref[...], 0.0)
    z = jnp.dot(h, w2_ref[...], preferred_element_type=jnp.float32) + b2_ref[...]
    p = jnp.where(z >= 0, z, pa_ref[...] * z)
    out_ref[...] = jnp.dot(p, w3_ref[...],
                           preferred_element_type=jnp.float32) + b3_ref[...]


@jax.jit
def kernel(x, adj, W1, b1, W2, b2, prelu_a, W3, b3):
    n, d = x.shape
    nhid = W2.shape[0]
    nclass = W3.shape[1]

    x_res = x.astype(jnp.bfloat16)
    w1a = W1[:d]
    w1b = W1[d:]
    b1r = b1.reshape(1, nhid)
    b2r = b2.reshape(1, nhid)
    par = prelu_a.reshape(1, nhid)
    b3r = b3.reshape(1, nclass)

    grid = (_NSTEP,)
    out = pl.pallas_call(
        _body,
        grid=grid,
        in_specs=[
            pl.BlockSpec(memory_space=pltpu.MemorySpace.HBM),
            pl.BlockSpec(memory_space=pltpu.MemorySpace.HBM),
            pl.BlockSpec((n, d), lambda i: (0, 0)),
            pl.BlockSpec((d, nhid), lambda i: (0, 0)),
            pl.BlockSpec((d, nhid), lambda i: (0, 0)),
            pl.BlockSpec((1, nhid), lambda i: (0, 0)),
            pl.BlockSpec((nhid, nhid), lambda i: (0, 0)),
            pl.BlockSpec((1, nhid), lambda i: (0, 0)),
            pl.BlockSpec((1, nhid), lambda i: (0, 0)),
            pl.BlockSpec((nhid, nclass), lambda i: (0, 0)),
            pl.BlockSpec((1, nclass), lambda i: (0, 0)),
        ],
        out_specs=pl.BlockSpec((_BM, nclass), lambda i: (i, 0)),
        out_shape=jax.ShapeDtypeStruct((n, nclass), jnp.float32),
        scratch_shapes=[
            pltpu.VMEM((_NBUF, _BR, _N), jnp.float32),
            pltpu.VMEM((_NBUF, _BR, _N), jnp.float32),
            pltpu.SemaphoreType.DMA((_NBUF,)),
            pltpu.SemaphoreType.DMA((_NBUF,)),
        ],
        compiler_params=pltpu.CompilerParams(
            dimension_semantics=("arbitrary",),
            vmem_limit_bytes=128 * 1024 * 1024),
    )(adj, adj, x_res, w1a, w1b, b1r, W2, b2r, par, W3, b3r)
    return out


# speedup vs baseline: 1.2333x; 1.0553x over previous
import functools

import jax
import jax.numpy as jnp
from jax.experimental import pallas as pl
from jax.experimental.pallas import tpu as pltpu

_BM = 400
_NS = 2
_BH = _BM // _NS


def _body(adj0_ref, adj1_ref,
          xk_ref, w1a_ref, w1b_ref, b1_ref,
          w2_ref, b2_ref, pa_ref, w3_ref, b3_ref, out_ref):
    i = pl.program_id(0)
    aggs = [jnp.dot(a[...], xk_ref[...], preferred_element_type=jnp.float32)
            for a in (adj0_ref, adj1_ref)]
    agg = jnp.concatenate(aggs, axis=0)
    xr = xk_ref[pl.ds(i * _BM, _BM), :]
    h = jnp.dot(xr, w1a_ref[...], preferred_element_type=jnp.float32)
    h += jnp.dot(agg, w1b_ref[...], preferred_element_type=jnp.float32)
    h = jnp.maximum(h + b1_ref[...], 0.0)
    z = jnp.dot(h, w2_ref[...], preferred_element_type=jnp.float32) + b2_ref[...]
    p = jnp.where(z >= 0, z, pa_ref[...] * z)
    out_ref[...] = jnp.dot(p, w3_ref[...],
                           preferred_element_type=jnp.float32) + b3_ref[...]


@jax.jit
def kernel(x, adj, W1, b1, W2, b2, prelu_a, W3, b3):
    n, d = x.shape
    nhid = W2.shape[0]
    nclass = W3.shape[1]

    w1a = W1[:d]
    w1b = W1[d:]
    b1r = b1.reshape(1, nhid)
    b2r = b2.reshape(1, nhid)
    par = prelu_a.reshape(1, nhid)
    b3r = b3.reshape(1, nclass)

    grid = (n // _BM,)
    out = pl.pallas_call(
        _body,
        grid=grid,
        in_specs=[
            *[pl.BlockSpec((_BH, n), functools.partial(
                lambda s, i: (_NS * i + s, 0), s))
              for s in range(_NS)],
            pl.BlockSpec((n, d), lambda i: (0, 0)),
            pl.BlockSpec((d, nhid), lambda i: (0, 0)),
            pl.BlockSpec((d, nhid), lambda i: (0, 0)),
            pl.BlockSpec((1, nhid), lambda i: (0, 0)),
            pl.BlockSpec((nhid, nhid), lambda i: (0, 0)),
            pl.BlockSpec((1, nhid), lambda i: (0, 0)),
            pl.BlockSpec((1, nhid), lambda i: (0, 0)),
            pl.BlockSpec((nhid, nclass), lambda i: (0, 0)),
            pl.BlockSpec((1, nclass), lambda i: (0, 0)),
        ],
        out_specs=pl.BlockSpec((_BM, nclass), lambda i: (i, 0)),
        out_shape=jax.ShapeDtypeStruct((n, nclass), jnp.float32),
        compiler_params=pltpu.CompilerParams(
            dimension_semantics=("parallel",),
            vmem_limit_bytes=128 * 1024 * 1024),
    )(*([adj] * _NS), x, w1a, w1b, b1r, W2, b2r, par, W3, b3r)
    return out
